# Initial kernel scaffold; baseline (speedup 1.0000x reference)
import jax
import jax.numpy as jnp
from jax.experimental import pallas as pl


def kernel(P, d_hw, circuit_edge_pairs, circuit_edge_weights):
    raise NotImplementedError("write your pallas kernel here")


# speedup vs baseline: 18.8942x; 18.8942x over previous
import functools

import jax
import jax.numpy as jnp
from jax import lax
from jax.experimental import pallas as pl
from jax.experimental.pallas import tpu as pltpu
from jax.experimental.pallas import tpu_sc as plsc

NC = 2
NS = 16
L = 16
NW = NC * NS


def _matmul_kernel(p_ref, d_ref, out_ref):
    a_hw = (d_ref[...] == 1).astype(jnp.float32)
    out_ref[...] = jnp.dot(p_ref[...], a_hw, preferred_element_type=jnp.float32)


def _compute_pa(p_flat, d_hw, block_rows):
    rows = p_flat.shape[0]
    q = p_flat.shape[1]
    grid = rows // block_rows
    return pl.pallas_call(
        _matmul_kernel,
        grid=(grid,),
        in_specs=[
            pl.BlockSpec((block_rows, q), lambda i: (i, 0)),
            pl.BlockSpec((q, q), lambda i: (0, 0)),
        ],
        out_specs=pl.BlockSpec((block_rows, q), lambda i: (i, 0)),
        out_shape=jax.ShapeDtypeStruct((rows, q), jnp.float32),
    )(p_flat, d_hw)


def _finalize_kernel(s_ref, w_ref, o_ref, *, wps, b):
    total = jnp.float32(0.0)
    for bb in range(b):
        sb = jnp.sum(s_ref[bb * wps:(bb + 1) * wps, :])
        wb = jnp.maximum(jnp.sum(w_ref[bb * wps:(bb + 1) * wps, :]), 1e-8)
        total = total + sb / wb
    o_ref[0, 0] = -total / b


def _make_sc_kernel(q, epw, cs, ncs, tail):
    qc = q // L
    mesh = plsc.VectorSubcoreMesh(
        core_axis_name="c", subcore_axis_name="s", num_cores=NC, num_subcores=NS)

    def body(pa_hbm, p_hbm, i_hbm, j_hbm, w_hbm, s_out, w_out,
             idx_i, idx_j, wv, ri, rj, stage, sem_i, sem_j):
        cid = lax.axis_index("c")
        sid = lax.axis_index("s")
        wid = sid * NC + cid
        base = wid * epw

        def load_chunk(off, n):
            pltpu.sync_copy(i_hbm.at[pl.ds(off, n)], idx_i.at[pl.ds(0, n)])
            pltpu.sync_copy(j_hbm.at[pl.ds(off, n)], idx_j.at[pl.ds(0, n)])
            pltpu.sync_copy(w_hbm.at[pl.ds(off, n)], wv.at[pl.ds(0, n)])
            cp_i = pltpu.async_copy(
                pa_hbm.at[idx_i.at[pl.ds(0, n)]], ri.at[pl.ds(0, n)], sem_i)
            cp_j = pltpu.async_copy(
                p_hbm.at[idx_j.at[pl.ds(0, n)]], rj.at[pl.ds(0, n)], sem_j)
            cp_i.wait()
            cp_j.wait()

        def accum_chunk(n, carry):
            def group_body(g, carry):
                accs, wacc = carry
                w16 = wv[pl.ds(g * L, L)]
                wacc = wacc + w16
                accs = list(accs)
                for k in range(L):
                    e = g * L + k
                    wspl = w16[k]
                    for c in range(qc):
                        pi = ri[e, pl.ds(c * L, L)]
                        pj = rj[e, pl.ds(c * L, L)]
                        accs[c] = accs[c] + pi * pj * wspl
                return tuple(accs), wacc

            return lax.fori_loop(0, n // L, group_body, carry)

        zero = jnp.zeros((L,), jnp.float32)
        carry0 = (tuple(zero for _ in range(qc)), zero)

        def chunk_body(g, carry):
            load_chunk(base + g * cs, cs)
            return accum_chunk(cs, carry)

        carry = lax.fori_loop(0, ncs, chunk_body, carry0)
        if tail:
            load_chunk(base + ncs * cs, tail)
            carry = accum_chunk(tail, carry)

        accs, wacc = carry
        stot = accs[0]
        for c in range(1, qc):
            stot = stot + accs[c]
        stage[pl.ds(0, L)] = stot
        stage[pl.ds(L, L)] = wacc
        pltpu.sync_copy(stage.at[pl.ds(0, L)], s_out.at[pl.ds(wid * L, L)])
        pltpu.sync_copy(stage.at[pl.ds(L, L)], w_out.at[pl.ds(wid * L, L)])

    return pl.kernel(
        body,
        out_type=(
            jax.ShapeDtypeStruct((NW * L,), jnp.float32),
            jax.ShapeDtypeStruct((NW * L,), jnp.float32),
        ),
        mesh=mesh,
        scratch_types=[
            pltpu.VMEM((cs,), jnp.int32),
            pltpu.VMEM((cs,), jnp.int32),
            pltpu.VMEM((cs,), jnp.float32),
            pltpu.VMEM((cs, q), jnp.float32),
            pltpu.VMEM((cs, q), jnp.float32),
            pltpu.VMEM((2 * L,), jnp.float32),
            pltpu.SemaphoreType.DMA,
            pltpu.SemaphoreType.DMA,
        ],
    )


def kernel(P, d_hw, circuit_edge_pairs, circuit_edge_weights):
    b, n, q = P.shape
    e = circuit_edge_pairs.shape[1]

    p_flat = P.reshape(b * n, q)
    offs = (jnp.arange(b, dtype=jnp.int32) * n)[:, None]
    i_flat = (circuit_edge_pairs[:, :, 0] + offs).reshape(b * e)
    j_flat = (circuit_edge_pairs[:, :, 1] + offs).reshape(b * e)
    w_flat = circuit_edge_weights.reshape(b * e)

    pa_flat = _compute_pa(p_flat, d_hw, block_rows=1000)

    wps = NW // b
    epw = e // wps
    cs = 128
    ncs = epw // cs
    tail = epw - ncs * cs
    sc = _make_sc_kernel(q, epw, cs, ncs, tail)
    s_part, w_part = sc(pa_flat, p_flat, i_flat, j_flat, w_flat)
    s_part = s_part.reshape(NW, L)
    w_part = w_part.reshape(NW, L)

    fin = pl.pallas_call(
        functools.partial(_finalize_kernel, wps=wps, b=b),
        in_specs=[
            pl.BlockSpec(memory_space=pltpu.VMEM),
            pl.BlockSpec(memory_space=pltpu.VMEM),
        ],
        out_specs=pl.BlockSpec(memory_space=pltpu.SMEM),
        out_shape=jax.ShapeDtypeStruct((1, 1), jnp.float32),
    )(s_part, w_part)
    return fin[0, 0]
